# Initial kernel scaffold; baseline (speedup 1.0000x reference)
import functools

import jax
import jax.numpy as jnp
from jax import lax
from jax.experimental import pallas as pl
from jax.experimental.pallas import tpu as pltpu
from jax.experimental.pallas import tpu_sc as plsc

N = 10000
E = 320000
H = 128
NC = 2
NS = 16
NT = NC * NS
NP = 10240
EPT = E // NT
CH = 80
NCHT = EPT // CH
RB = 400
GRID = N // RB

_mesh = plsc.VectorSubcoreMesh(core_axis_name="c", subcore_axis_name="s")


@functools.partial(
    pl.kernel,
    out_type=jax.ShapeDtypeStruct((NT, NP), jnp.float32),
    mesh=_mesh,
    scratch_types=[
        pltpu.VMEM((EPT,), jnp.int32),
        pltpu.VMEM((NP,), jnp.float32),
    ],
)
def _sc_degree(dst_hbm, out_hbm, dst_v, hist_v):
    cid = lax.axis_index("c")
    sid = lax.axis_index("s")
    wid = cid * NS + sid
    zero = jnp.zeros((16,), jnp.float32)

    def zbody(i, carry):
        hist_v[pl.ds(i * 16, 16)] = zero
        return carry

    lax.fori_loop(0, NP // 16, zbody, 0)
    pltpu.sync_copy(dst_hbm.at[pl.ds(wid * EPT, EPT)], dst_v)
    ones = jnp.ones((16,), jnp.float32)

    def body(i, carry):
        idx = dst_v[pl.ds(i * 16, 16)]
        plsc.addupdate_scatter(hist_v, [idx], ones)
        return carry

    lax.fori_loop(0, EPT // 16, body, 0)
    pltpu.sync_copy(hist_v, out_hbm.at[wid])


@functools.partial(
    pl.kernel,
    out_type=jax.ShapeDtypeStruct((NC, NP, H), jnp.float32),
    mesh=_mesh,
    scratch_types=[
        pltpu.VMEM((NCHT, CH), jnp.int32),
        pltpu.VMEM((NCHT, CH), jnp.int32),
        pltpu.VMEM((CH, H), jnp.float32),
        pltpu.VMEM((128, H), jnp.float32),
        pltpu.VMEM_SHARED((NP, H), jnp.float32),
        pltpu.SemaphoreType.DMA,
    ],
)
def _sc_aggregate(hp_hbm, src_hbm, dst_hbm, out_hbm,
                  sidx_v, didx_v, rows_v, zb_v, acc_sh, sem):
    cid = lax.axis_index("c")
    sid = lax.axis_index("s")
    wid = cid * NS + sid
    zero = jnp.zeros((16,), jnp.float32)

    def zbody(i, carry):
        zb_v[i >> 3, pl.ds((i & 7) * 16, 16)] = zero
        return carry

    lax.fori_loop(0, 128 * (H // 16), zbody, 0)
    base = sid * (NP // NS)
    for j in range(NP // NS // 128):
        pltpu.sync_copy(zb_v, acc_sh.at[pl.ds(base + j * 128, 128)])
    plsc.subcore_barrier()

    pltpu.sync_copy(src_hbm.at[pl.ds(wid * NCHT, NCHT)], sidx_v)
    pltpu.sync_copy(dst_hbm.at[pl.ds(wid * NCHT, NCHT)], didx_v)

    def body(j, carry):
        pltpu.async_copy(hp_hbm.at[sidx_v.at[j]], rows_v, sem).wait()
        pltpu.sync_copy(rows_v, acc_sh.at[didx_v.at[j]], add=True)
        return carry

    lax.fori_loop(0, NCHT, body, 0)
    plsc.subcore_barrier()
    for j in range(NP // NS // 128):
        pltpu.sync_copy(acc_sh.at[pl.ds(base + j * 128, 128)],
                        out_hbm.at[cid, pl.ds(base + j * 128, 128)])


def _dot(a, b):
    return jnp.dot(a, b, preferred_element_type=jnp.float32,
                   precision=jax.lax.Precision.HIGHEST)


def _tc_first_body(x_ref, w_ref, degT_ref, hp_ref, dinv_ref):
    deg = jnp.sum(degT_ref[...], axis=1, keepdims=True) + 1.0
    dinv = lax.rsqrt(deg)
    h = _dot(x_ref[...], w_ref[...])
    hp_ref[...] = h * dinv
    dinv_ref[...] = dinv


def _tc_first(x, W, degT):
    return pl.pallas_call(
        _tc_first_body,
        grid=(GRID,),
        in_specs=[pl.BlockSpec((RB, H), lambda i: (i, 0)),
                  pl.BlockSpec((H, H), lambda i: (0, 0)),
                  pl.BlockSpec((RB, NT), lambda i: (i, 0))],
        out_specs=[pl.BlockSpec((RB, H), lambda i: (i, 0)),
                   pl.BlockSpec((RB, 1), lambda i: (i, 0))],
        out_shape=[jax.ShapeDtypeStruct((N, H), jnp.float32),
                   jax.ShapeDtypeStruct((N, 1), jnp.float32)],
    )(x, W, degT)


def _tc_post_body(agg_ref, hp_ref, dinv_ref, b_ref, z_ref, stats_ref):
    i = pl.program_id(0)
    s = agg_ref[0] + agg_ref[1] + hp_ref[...]
    z = jnp.maximum(s * dinv_ref[...] + b_ref[...], 0.0)
    z_ref[...] = z

    @pl.when(i == 0)
    def _():
        stats_ref[...] = jnp.zeros_like(stats_ref)

    stats_ref[...] += jnp.concatenate(
        [jnp.sum(z, axis=0, keepdims=True),
         jnp.sum(z * z, axis=0, keepdims=True)], axis=0)


def _tc_post(agg, hp, dinv, b):
    return pl.pallas_call(
        _tc_post_body,
        grid=(GRID,),
        in_specs=[pl.BlockSpec((NC, RB, H), lambda i: (0, i, 0)),
                  pl.BlockSpec((RB, H), lambda i: (i, 0)),
                  pl.BlockSpec((RB, 1), lambda i: (i, 0)),
                  pl.BlockSpec((1, H), lambda i: (0, 0))],
        out_specs=[pl.BlockSpec((RB, H), lambda i: (i, 0)),
                   pl.BlockSpec((2, H), lambda i: (0, 0))],
        out_shape=[jax.ShapeDtypeStruct((N, H), jnp.float32),
                   jax.ShapeDtypeStruct((2, H), jnp.float32)],
    )(agg, hp, dinv, b)


def _bn_coeffs(stats, g, be):
    m = stats[0:1, :] * (1.0 / N)
    v = stats[1:2, :] * (1.0 / N) - m * m
    a = g * lax.rsqrt(v + 1e-5)
    return a, be - m * a


def _tc_bnmm_body(z_ref, stats_ref, g_ref, be_ref, w_ref, dinv_ref, hp_ref):
    a, c = _bn_coeffs(stats_ref[...], g_ref[...], be_ref[...])
    h = _dot(z_ref[...] * a, w_ref[...]) + _dot(c, w_ref[...])
    hp_ref[...] = h * dinv_ref[...]


def _tc_bnmm(z, stats, g, be, W, dinv):
    return pl.pallas_call(
        _tc_bnmm_body,
        grid=(GRID,),
        in_specs=[pl.BlockSpec((RB, H), lambda i: (i, 0)),
                  pl.BlockSpec((2, H), lambda i: (0, 0)),
                  pl.BlockSpec((1, H), lambda i: (0, 0)),
                  pl.BlockSpec((1, H), lambda i: (0, 0)),
                  pl.BlockSpec((H, H), lambda i: (0, 0)),
                  pl.BlockSpec((RB, 1), lambda i: (i, 0))],
        out_specs=pl.BlockSpec((RB, H), lambda i: (i, 0)),
        out_shape=jax.ShapeDtypeStruct((N, H), jnp.float32),
    )(z, stats, g, be, W, dinv)


def _tc_final_body(z_ref, stats_ref, g_ref, be_ref, w_ref, bl_ref, o_ref):
    a, c = _bn_coeffs(stats_ref[...], g_ref[...], be_ref[...])
    t = _dot(z_ref[...] * a, w_ref[...]) + _dot(c, w_ref[...]) + bl_ref[...]
    r = jnp.maximum(t, 0.0)
    e = jnp.exp(r - jnp.max(r, axis=1, keepdims=True))
    o_ref[...] = e / jnp.sum(e, axis=1, keepdims=True)


def _tc_final(z, stats, g, be, W, bl):
    return pl.pallas_call(
        _tc_final_body,
        grid=(GRID,),
        in_specs=[pl.BlockSpec((RB, H), lambda i: (i, 0)),
                  pl.BlockSpec((2, H), lambda i: (0, 0)),
                  pl.BlockSpec((1, H), lambda i: (0, 0)),
                  pl.BlockSpec((1, H), lambda i: (0, 0)),
                  pl.BlockSpec((H, H), lambda i: (0, 0)),
                  pl.BlockSpec((1, H), lambda i: (0, 0))],
        out_specs=pl.BlockSpec((RB, H), lambda i: (i, 0)),
        out_shape=jax.ShapeDtypeStruct((N, H), jnp.float32),
    )(z, stats, g, be, W, bl)


def kernel(x, edge_index, W1, b1, g1, be1, W2, b2, g2, be2,
           W3, b3, g3, be3, Wl, bl):
    src = edge_index[0].reshape(NT * NCHT, CH)
    dst_flat = edge_index[1]
    dst = dst_flat.reshape(NT * NCHT, CH)

    degs = _sc_degree(dst_flat)
    degT = degs.T

    b1r, g1r, be1r = b1.reshape(1, H), g1.reshape(1, H), be1.reshape(1, H)
    b2r, g2r, be2r = b2.reshape(1, H), g2.reshape(1, H), be2.reshape(1, H)
    b3r, g3r, be3r = b3.reshape(1, H), g3.reshape(1, H), be3.reshape(1, H)
    blr = bl.reshape(1, H)

    hp, dinv = _tc_first(x, W1, degT)

    agg = _sc_aggregate(hp, src, dst)
    z1, s1 = _tc_post(agg, hp, dinv, b1r)
    hp = _tc_bnmm(z1, s1, g1r, be1r, W2, dinv)

    agg = _sc_aggregate(hp, src, dst)
    z2, s2 = _tc_post(agg, hp, dinv, b2r)
    hp = _tc_bnmm(z2, s2, g2r, be2r, W3, dinv)

    agg = _sc_aggregate(hp, src, dst)
    z3, s3 = _tc_post(agg, hp, dinv, b3r)
    return _tc_final(z3, s3, g3r, be3r, Wl, blr)


# speedup vs baseline: 15.0000x; 15.0000x over previous
import functools

import jax
import jax.numpy as jnp
from jax import lax
from jax.experimental import pallas as pl
from jax.experimental.pallas import tpu as pltpu
from jax.experimental.pallas import tpu_sc as plsc

N = 10000
E = 320000
H = 128
NC = 2
NS = 16
NT = NC * NS
NP = 10240
EPT = E // NT
CH = 80
NCHT = EPT // CH
RB = 400
GRID = N // RB


@functools.cache
def _sc_kernels():
    mesh = plsc.VectorSubcoreMesh(core_axis_name="c", subcore_axis_name="s",
                                  num_cores=NC, num_subcores=NS)

    @functools.partial(
        pl.kernel,
        out_type=jax.ShapeDtypeStruct((NT, NP), jnp.float32),
        mesh=mesh,
        compiler_params=pltpu.CompilerParams(needs_layout_passes=False),
        scratch_types=[
            pltpu.VMEM((EPT,), jnp.int32),
            pltpu.VMEM((NP,), jnp.float32),
        ],
    )
    def _sc_degree(dst_hbm, out_hbm, dst_v, hist_v):
        cid = lax.axis_index("c")
        sid = lax.axis_index("s")
        wid = cid * NS + sid
        zero = jnp.zeros((16,), jnp.float32)

        def zbody(i, carry):
            hist_v[pl.ds(i * 16, 16)] = zero
            return carry

        lax.fori_loop(0, NP // 16, zbody, 0)
        pltpu.sync_copy(dst_hbm.at[pl.ds(wid * EPT, EPT)], dst_v)
        ones = jnp.ones((16,), jnp.float32)

        def body(i, carry):
            idx = dst_v[pl.ds(i * 16, 16)]
            plsc.addupdate_scatter(hist_v, [idx], ones)
            return carry

        lax.fori_loop(0, EPT // 16, body, 0)
        pltpu.sync_copy(hist_v, out_hbm.at[wid])

    @functools.partial(
        pl.kernel,
        out_type=jax.ShapeDtypeStruct((NC, NP, H), jnp.float32),
        mesh=mesh,
        compiler_params=pltpu.CompilerParams(needs_layout_passes=False,
                                             use_tc_tiling_on_sc=False),
        scratch_types=[
            pltpu.VMEM((NCHT, CH), jnp.int32),
            pltpu.VMEM((NCHT, CH), jnp.int32),
            pltpu.VMEM((CH, H), jnp.float32),
            pltpu.VMEM((128, H), jnp.float32),
            pltpu.VMEM_SHARED((NP, H), jnp.float32),
            pltpu.SemaphoreType.DMA,
        ],
    )
    def _sc_aggregate(hp_hbm, src_hbm, dst_hbm, out_hbm,
                      sidx_v, didx_v, rows_v, zb_v, acc_sh, sem):
        cid = lax.axis_index("c")
        sid = lax.axis_index("s")
        wid = cid * NS + sid
        zero = jnp.zeros((16,), jnp.float32)

        def zbody(i, carry):
            zb_v[i >> 3, pl.ds((i & 7) * 16, 16)] = zero
            return carry

        lax.fori_loop(0, 128 * (H // 16), zbody, 0)
        base = sid * (NP // NS)
        for j in range(NP // NS // 128):
            pltpu.sync_copy(zb_v, acc_sh.at[pl.ds(base + j * 128, 128)])
        plsc.subcore_barrier()

        pltpu.sync_copy(src_hbm.at[pl.ds(wid * NCHT, NCHT)], sidx_v)
        pltpu.sync_copy(dst_hbm.at[pl.ds(wid * NCHT, NCHT)], didx_v)

        def body(j, carry):
            pltpu.async_copy(hp_hbm.at[sidx_v.at[j]], rows_v, sem).wait()
            pltpu.sync_copy(rows_v, acc_sh.at[didx_v.at[j]], add=True)
            return carry

        lax.fori_loop(0, NCHT, body, 0)
        plsc.subcore_barrier()
        for j in range(NP // NS // 128):
            pltpu.sync_copy(acc_sh.at[pl.ds(base + j * 128, 128)],
                            out_hbm.at[cid, pl.ds(base + j * 128, 128)])

    return _sc_degree, _sc_aggregate


def _dot(a, b):
    return jnp.dot(a, b, preferred_element_type=jnp.float32,
                   precision=jax.lax.Precision.HIGHEST)


def _tc_first_body(x_ref, w_ref, degT_ref, hp_ref, dinv_ref):
    deg = jnp.sum(degT_ref[...], axis=1, keepdims=True) + 1.0
    dinv = lax.rsqrt(deg)
    h = _dot(x_ref[...], w_ref[...])
    hp_ref[...] = h * dinv
    dinv_ref[...] = dinv


def _tc_first(x, W, degT):
    return pl.pallas_call(
        _tc_first_body,
        grid=(GRID,),
        in_specs=[pl.BlockSpec((RB, H), lambda i: (i, 0)),
                  pl.BlockSpec((H, H), lambda i: (0, 0)),
                  pl.BlockSpec((RB, NT), lambda i: (i, 0))],
        out_specs=[pl.BlockSpec((RB, H), lambda i: (i, 0)),
                   pl.BlockSpec((RB, 1), lambda i: (i, 0))],
        out_shape=[jax.ShapeDtypeStruct((N, H), jnp.float32),
                   jax.ShapeDtypeStruct((N, 1), jnp.float32)],
    )(x, W, degT)


def _tc_post_body(agg_ref, hp_ref, dinv_ref, b_ref, z_ref, stats_ref):
    i = pl.program_id(0)
    s = agg_ref[0] + agg_ref[1] + hp_ref[...]
    z = jnp.maximum(s * dinv_ref[...] + b_ref[...], 0.0)
    z_ref[...] = z

    @pl.when(i == 0)
    def _():
        stats_ref[...] = jnp.zeros_like(stats_ref)

    stats_ref[...] += jnp.concatenate(
        [jnp.sum(z, axis=0, keepdims=True),
         jnp.sum(z * z, axis=0, keepdims=True)], axis=0)


def _tc_post(agg, hp, dinv, b):
    return pl.pallas_call(
        _tc_post_body,
        grid=(GRID,),
        in_specs=[pl.BlockSpec((NC, RB, H), lambda i: (0, i, 0)),
                  pl.BlockSpec((RB, H), lambda i: (i, 0)),
                  pl.BlockSpec((RB, 1), lambda i: (i, 0)),
                  pl.BlockSpec((1, H), lambda i: (0, 0))],
        out_specs=[pl.BlockSpec((RB, H), lambda i: (i, 0)),
                   pl.BlockSpec((2, H), lambda i: (0, 0))],
        out_shape=[jax.ShapeDtypeStruct((N, H), jnp.float32),
                   jax.ShapeDtypeStruct((2, H), jnp.float32)],
    )(agg, hp, dinv, b)


def _bn_coeffs(stats, g, be):
    m = stats[0:1, :] * (1.0 / N)
    v = stats[1:2, :] * (1.0 / N) - m * m
    a = g * lax.rsqrt(v + 1e-5)
    return a, be - m * a


def _tc_bnmm_body(z_ref, stats_ref, g_ref, be_ref, w_ref, dinv_ref, hp_ref):
    a, c = _bn_coeffs(stats_ref[...], g_ref[...], be_ref[...])
    h = _dot(z_ref[...] * a, w_ref[...]) + _dot(c, w_ref[...])
    hp_ref[...] = h * dinv_ref[...]


def _tc_bnmm(z, stats, g, be, W, dinv):
    return pl.pallas_call(
        _tc_bnmm_body,
        grid=(GRID,),
        in_specs=[pl.BlockSpec((RB, H), lambda i: (i, 0)),
                  pl.BlockSpec((2, H), lambda i: (0, 0)),
                  pl.BlockSpec((1, H), lambda i: (0, 0)),
                  pl.BlockSpec((1, H), lambda i: (0, 0)),
                  pl.BlockSpec((H, H), lambda i: (0, 0)),
                  pl.BlockSpec((RB, 1), lambda i: (i, 0))],
        out_specs=pl.BlockSpec((RB, H), lambda i: (i, 0)),
        out_shape=jax.ShapeDtypeStruct((N, H), jnp.float32),
    )(z, stats, g, be, W, dinv)


def _tc_final_body(z_ref, stats_ref, g_ref, be_ref, w_ref, bl_ref, o_ref):
    a, c = _bn_coeffs(stats_ref[...], g_ref[...], be_ref[...])
    t = _dot(z_ref[...] * a, w_ref[...]) + _dot(c, w_ref[...]) + bl_ref[...]
    r = jnp.maximum(t, 0.0)
    e = jnp.exp(r - jnp.max(r, axis=1, keepdims=True))
    o_ref[...] = e / jnp.sum(e, axis=1, keepdims=True)


def _tc_final(z, stats, g, be, W, bl):
    return pl.pallas_call(
        _tc_final_body,
        grid=(GRID,),
        in_specs=[pl.BlockSpec((RB, H), lambda i: (i, 0)),
                  pl.BlockSpec((2, H), lambda i: (0, 0)),
                  pl.BlockSpec((1, H), lambda i: (0, 0)),
                  pl.BlockSpec((1, H), lambda i: (0, 0)),
                  pl.BlockSpec((H, H), lambda i: (0, 0)),
                  pl.BlockSpec((1, H), lambda i: (0, 0))],
        out_specs=pl.BlockSpec((RB, H), lambda i: (i, 0)),
        out_shape=jax.ShapeDtypeStruct((N, H), jnp.float32),
    )(z, stats, g, be, W, bl)


def kernel(x, edge_index, W1, b1, g1, be1, W2, b2, g2, be2,
           W3, b3, g3, be3, Wl, bl):
    _sc_degree, _sc_aggregate = _sc_kernels()
    src = edge_index[0].reshape(NT * NCHT, CH)
    dst_flat = edge_index[1]
    dst = dst_flat.reshape(NT * NCHT, CH)

    degs = _sc_degree(dst_flat)
    degT = degs.T

    b1r, g1r, be1r = b1.reshape(1, H), g1.reshape(1, H), be1.reshape(1, H)
    b2r, g2r, be2r = b2.reshape(1, H), g2.reshape(1, H), be2.reshape(1, H)
    b3r, g3r, be3r = b3.reshape(1, H), g3.reshape(1, H), be3.reshape(1, H)
    blr = bl.reshape(1, H)

    hp, dinv = _tc_first(x, W1, degT)

    agg = _sc_aggregate(hp, src, dst)
    z1, s1 = _tc_post(agg, hp, dinv, b1r)
    hp = _tc_bnmm(z1, s1, g1r, be1r, W2, dinv)

    agg = _sc_aggregate(hp, src, dst)
    z2, s2 = _tc_post(agg, hp, dinv, b2r)
    hp = _tc_bnmm(z2, s2, g2r, be2r, W3, dinv)

    agg = _sc_aggregate(hp, src, dst)
    z3, s3 = _tc_post(agg, hp, dinv, b3r)
    return _tc_final(z3, s3, g3r, be3r, Wl, blr)


# speedup vs baseline: 23.2824x; 1.5522x over previous
import functools

import jax
import jax.numpy as jnp
from jax import lax
from jax.experimental import pallas as pl
from jax.experimental.pallas import tpu as pltpu
from jax.experimental.pallas import tpu_sc as plsc

N = 10000
E = 320000
H = 128
NC = 2
NS = 16
NT = NC * NS
NP = 10240
EPT = E // NT
CH = 100
NCHT = EPT // CH
RB = 400
GRID = N // RB


@functools.cache
def _sc_kernels():
    mesh = plsc.VectorSubcoreMesh(core_axis_name="c", subcore_axis_name="s",
                                  num_cores=NC, num_subcores=NS)

    @functools.partial(
        pl.kernel,
        out_type=jax.ShapeDtypeStruct((NT, NP), jnp.float32),
        mesh=mesh,
        compiler_params=pltpu.CompilerParams(needs_layout_passes=False),
        scratch_types=[
            pltpu.VMEM((EPT,), jnp.int32),
            pltpu.VMEM((NP,), jnp.float32),
        ],
    )
    def _sc_degree(dst_hbm, out_hbm, dst_v, hist_v):
        cid = lax.axis_index("c")
        sid = lax.axis_index("s")
        wid = cid * NS + sid
        zero = jnp.zeros((16,), jnp.float32)

        def zbody(i, carry):
            hist_v[pl.ds(i * 16, 16)] = zero
            return carry

        lax.fori_loop(0, NP // 16, zbody, 0)
        pltpu.sync_copy(dst_hbm.at[pl.ds(wid * EPT, EPT)], dst_v)
        ones = jnp.ones((16,), jnp.float32)

        def body(i, carry):
            idx = dst_v[pl.ds(i * 16, 16)]
            plsc.addupdate_scatter(hist_v, [idx], ones)
            return carry

        lax.fori_loop(0, EPT // 16, body, 0)
        pltpu.sync_copy(hist_v, out_hbm.at[wid])

    @functools.partial(
        pl.kernel,
        out_type=jax.ShapeDtypeStruct((NC, NP, H), jnp.float32),
        mesh=mesh,
        compiler_params=pltpu.CompilerParams(needs_layout_passes=False,
                                             use_tc_tiling_on_sc=False),
        scratch_types=[
            pltpu.VMEM((NCHT, CH), jnp.int32),
            pltpu.VMEM((NCHT, CH), jnp.int32),
            pltpu.VMEM((CH, H), jnp.float32),
            pltpu.VMEM((CH, H), jnp.float32),
            pltpu.VMEM((16, H), jnp.float32),
            pltpu.VMEM_SHARED((NP, H), jnp.float32),
            pltpu.SemaphoreType.DMA,
            pltpu.SemaphoreType.DMA,
        ],
    )
    def _sc_aggregate(hp_hbm, src_hbm, dst_hbm, out_hbm,
                      sidx_v, didx_v, rows0_v, rows1_v, zb_v, acc_sh,
                      semA, semB):
        cid = lax.axis_index("c")
        sid = lax.axis_index("s")
        wid = cid * NS + sid
        zero = jnp.zeros((16,), jnp.float32)

        def zbody(i, carry):
            zb_v[i >> 3, pl.ds((i & 7) * 16, 16)] = zero
            return carry

        lax.fori_loop(0, 16 * (H // 16), zbody, 0)
        base = sid * (NP // NS)

        def zcpy(j, carry):
            pltpu.sync_copy(zb_v, acc_sh.at[pl.ds(base + j * 16, 16)])
            return carry

        lax.fori_loop(0, NP // NS // 16, zcpy, 0)
        plsc.subcore_barrier()

        pltpu.sync_copy(src_hbm.at[pl.ds(wid * NCHT, NCHT)], sidx_v)
        pltpu.sync_copy(dst_hbm.at[pl.ds(wid * NCHT, NCHT)], didx_v)

        def _wait(buf, sem):
            pltpu.make_async_copy(hp_hbm.at[pl.ds(0, CH)], buf, sem).wait()

        pltpu.async_copy(hp_hbm.at[sidx_v.at[0]], rows0_v, semA)

        def body(i, carry):
            j = 2 * i
            pltpu.async_copy(hp_hbm.at[sidx_v.at[j + 1]], rows1_v, semB)
            _wait(rows0_v, semA)
            pltpu.sync_copy(rows0_v, acc_sh.at[didx_v.at[j]], add=True)

            @pl.when(j + 2 < NCHT)
            def _():
                pltpu.async_copy(hp_hbm.at[sidx_v.at[j + 2]], rows0_v, semA)

            _wait(rows1_v, semB)
            pltpu.sync_copy(rows1_v, acc_sh.at[didx_v.at[j + 1]], add=True)
            return carry

        lax.fori_loop(0, NCHT // 2, body, 0)
        plsc.subcore_barrier()
        for j in range(NP // NS // 128):
            pltpu.sync_copy(acc_sh.at[pl.ds(base + j * 128, 128)],
                            out_hbm.at[cid, pl.ds(base + j * 128, 128)])

    return _sc_degree, _sc_aggregate


def _dot(a, b):
    return jnp.dot(a, b, preferred_element_type=jnp.float32,
                   precision=jax.lax.Precision.HIGHEST)


def _tc_first_body(x_ref, w_ref, degT_ref, hp_ref, dinv_ref):
    deg = jnp.sum(degT_ref[...], axis=1, keepdims=True) + 1.0
    dinv = lax.rsqrt(deg)
    h = _dot(x_ref[...], w_ref[...])
    hp_ref[...] = h * dinv
    dinv_ref[...] = dinv


def _tc_first(x, W, degT):
    return pl.pallas_call(
        _tc_first_body,
        grid=(GRID,),
        in_specs=[pl.BlockSpec((RB, H), lambda i: (i, 0)),
                  pl.BlockSpec((H, H), lambda i: (0, 0)),
                  pl.BlockSpec((RB, NT), lambda i: (i, 0))],
        out_specs=[pl.BlockSpec((RB, H), lambda i: (i, 0)),
                   pl.BlockSpec((RB, 1), lambda i: (i, 0))],
        out_shape=[jax.ShapeDtypeStruct((N, H), jnp.float32),
                   jax.ShapeDtypeStruct((N, 1), jnp.float32)],
    )(x, W, degT)


def _tc_post_body(agg_ref, hp_ref, dinv_ref, b_ref, z_ref, stats_ref):
    i = pl.program_id(0)
    s = agg_ref[0] + agg_ref[1] + hp_ref[...]
    z = jnp.maximum(s * dinv_ref[...] + b_ref[...], 0.0)
    z_ref[...] = z

    @pl.when(i == 0)
    def _():
        stats_ref[...] = jnp.zeros_like(stats_ref)

    stats_ref[...] += jnp.concatenate(
        [jnp.sum(z, axis=0, keepdims=True),
         jnp.sum(z * z, axis=0, keepdims=True)], axis=0)


def _tc_post(agg, hp, dinv, b):
    return pl.pallas_call(
        _tc_post_body,
        grid=(GRID,),
        in_specs=[pl.BlockSpec((NC, RB, H), lambda i: (0, i, 0)),
                  pl.BlockSpec((RB, H), lambda i: (i, 0)),
                  pl.BlockSpec((RB, 1), lambda i: (i, 0)),
                  pl.BlockSpec((1, H), lambda i: (0, 0))],
        out_specs=[pl.BlockSpec((RB, H), lambda i: (i, 0)),
                   pl.BlockSpec((2, H), lambda i: (0, 0))],
        out_shape=[jax.ShapeDtypeStruct((N, H), jnp.float32),
                   jax.ShapeDtypeStruct((2, H), jnp.float32)],
    )(agg, hp, dinv, b)


def _bn_coeffs(stats, g, be):
    m = stats[0:1, :] * (1.0 / N)
    v = stats[1:2, :] * (1.0 / N) - m * m
    a = g * lax.rsqrt(v + 1e-5)
    return a, be - m * a


def _tc_bnmm_body(z_ref, stats_ref, g_ref, be_ref, w_ref, dinv_ref, hp_ref):
    a, c = _bn_coeffs(stats_ref[...], g_ref[...], be_ref[...])
    h = _dot(z_ref[...] * a, w_ref[...]) + _dot(c, w_ref[...])
    hp_ref[...] = h * dinv_ref[...]


def _tc_bnmm(z, stats, g, be, W, dinv):
    return pl.pallas_call(
        _tc_bnmm_body,
        grid=(GRID,),
        in_specs=[pl.BlockSpec((RB, H), lambda i: (i, 0)),
                  pl.BlockSpec((2, H), lambda i: (0, 0)),
                  pl.BlockSpec((1, H), lambda i: (0, 0)),
                  pl.BlockSpec((1, H), lambda i: (0, 0)),
                  pl.BlockSpec((H, H), lambda i: (0, 0)),
                  pl.BlockSpec((RB, 1), lambda i: (i, 0))],
        out_specs=pl.BlockSpec((RB, H), lambda i: (i, 0)),
        out_shape=jax.ShapeDtypeStruct((N, H), jnp.float32),
    )(z, stats, g, be, W, dinv)


def _tc_final_body(z_ref, stats_ref, g_ref, be_ref, w_ref, bl_ref, o_ref):
    a, c = _bn_coeffs(stats_ref[...], g_ref[...], be_ref[...])
    t = _dot(z_ref[...] * a, w_ref[...]) + _dot(c, w_ref[...]) + bl_ref[...]
    r = jnp.maximum(t, 0.0)
    e = jnp.exp(r - jnp.max(r, axis=1, keepdims=True))
    o_ref[...] = e / jnp.sum(e, axis=1, keepdims=True)


def _tc_final(z, stats, g, be, W, bl):
    return pl.pallas_call(
        _tc_final_body,
        grid=(GRID,),
        in_specs=[pl.BlockSpec((RB, H), lambda i: (i, 0)),
                  pl.BlockSpec((2, H), lambda i: (0, 0)),
                  pl.BlockSpec((1, H), lambda i: (0, 0)),
                  pl.BlockSpec((1, H), lambda i: (0, 0)),
                  pl.BlockSpec((H, H), lambda i: (0, 0)),
                  pl.BlockSpec((1, H), lambda i: (0, 0))],
        out_specs=pl.BlockSpec((RB, H), lambda i: (i, 0)),
        out_shape=jax.ShapeDtypeStruct((N, H), jnp.float32),
    )(z, stats, g, be, W, bl)


def kernel(x, edge_index, W1, b1, g1, be1, W2, b2, g2, be2,
           W3, b3, g3, be3, Wl, bl):
    _sc_degree, _sc_aggregate = _sc_kernels()
    src = edge_index[0].reshape(NT * NCHT, CH)
    dst_flat = edge_index[1]
    dst = dst_flat.reshape(NT * NCHT, CH)

    degs = _sc_degree(dst_flat)
    degT = degs.T

    b1r, g1r, be1r = b1.reshape(1, H), g1.reshape(1, H), be1.reshape(1, H)
    b2r, g2r, be2r = b2.reshape(1, H), g2.reshape(1, H), be2.reshape(1, H)
    b3r, g3r, be3r = b3.reshape(1, H), g3.reshape(1, H), be3.reshape(1, H)
    blr = bl.reshape(1, H)

    hp, dinv = _tc_first(x, W1, degT)

    agg = _sc_aggregate(hp, src, dst)
    z1, s1 = _tc_post(agg, hp, dinv, b1r)
    hp = _tc_bnmm(z1, s1, g1r, be1r, W2, dinv)

    agg = _sc_aggregate(hp, src, dst)
    z2, s2 = _tc_post(agg, hp, dinv, b2r)
    hp = _tc_bnmm(z2, s2, g2r, be2r, W3, dinv)

    agg = _sc_aggregate(hp, src, dst)
    z3, s3 = _tc_post(agg, hp, dinv, b3r)
    return _tc_final(z3, s3, g3r, be3r, Wl, blr)
